# baseline (speedup 1.0000x reference)
import functools

import jax
import jax.numpy as jnp
from jax import lax
from jax.experimental import pallas as pl
from jax.experimental.pallas import tpu as pltpu
from jax.experimental.pallas import tpu_sc as plsc

B = 16384
C = 100
N = 1000000
LAM = 0.01

NC = 2
NS = 16

_BLK_A = 512


def _a_body(o_ref, lbl_ref, y_ref, y100_ref, ce_ref):
    o = o_ref[...]
    m = jnp.max(o, axis=1, keepdims=True)
    e = jnp.exp(o - m)
    s = jnp.sum(e, axis=1, keepdims=True)
    y = jnp.clip(e / s, 0.0001, 1.0 - 0.0001)
    y100_ref[...] = y
    y_ref[...] = jnp.concatenate(
        [y, jnp.zeros((o.shape[0], 128 - C), jnp.float32)], axis=1)
    lbl = lbl_ref[0, 0, :]
    cols = lax.broadcasted_iota(jnp.int32, o.shape, 1)
    pick = jnp.sum(jnp.where(cols == lbl[:, None], o, 0.0), axis=1)
    lse = m[:, 0] + jnp.log(s[:, 0])

    @pl.when(pl.program_id(0) == 0)
    def _():
        ce_ref[0, 0] = 0.0

    ce_ref[0, 0] += jnp.sum(pick - lse)


_softmax_ce = pl.pallas_call(
    _a_body,
    grid=(B // _BLK_A,),
    in_specs=[
        pl.BlockSpec((_BLK_A, C), lambda i: (i, 0)),
        pl.BlockSpec((1, 1, _BLK_A), lambda i: (i, 0, 0)),
    ],
    out_specs=[
        pl.BlockSpec((_BLK_A, 128), lambda i: (i, 0)),
        pl.BlockSpec((_BLK_A, C), lambda i: (i, 0)),
        pl.BlockSpec((1, 1), lambda i: (0, 0), memory_space=pltpu.SMEM),
    ],
    out_shape=[
        jax.ShapeDtypeStruct((B, 128), jnp.float32),
        jax.ShapeDtypeStruct((B, C), jnp.float32),
        jax.ShapeDtypeStruct((1, 1), jnp.float32),
    ],
)

_WROWS = B // 128 // NS
_NROUNDS = 5
_AUXN = N + 128


def _w_body(idx_hbm, w_hbm, aux_hbm, idx_v, jb_v, a_v, ip_v, sem):
    sid = lax.axis_index("s")
    base = sid * (_WROWS * 128)

    pltpu.sync_copy(idx_hbm.at[pl.ds(sid * _WROWS, _WROWS)], idx_v)
    for r in range(_WROWS):
        for c in range(8):
            jb_v[r, pl.ds(c * 16, 16)] = (
                base + r * 128 + c * 16
                + lax.broadcasted_iota(jnp.int32, (16,), 0)
            )

    cps = [pltpu.async_copy(jb_v.at[r], aux_hbm.at[idx_v.at[r]], sem)
           for r in range(_WROWS)]
    for cp in cps:
        cp.wait()
    plsc.subcore_barrier()
    cps = [pltpu.async_copy(aux_hbm.at[idx_v.at[r]], a_v.at[r], sem)
           for r in range(_WROWS)]
    for cp in cps:
        cp.wait()
    plsc.subcore_barrier()

    for _ in range(_NROUNDS):
        for r in range(_WROWS):
            for c in range(8):
                sl = pl.ds(c * 16, 16)
                ip_v[r, sl] = jnp.where(
                    jb_v[r, sl] > a_v[r, sl], idx_v[r, sl], jnp.int32(N))
        cps = [pltpu.async_copy(jb_v.at[r], aux_hbm.at[ip_v.at[r]], sem)
               for r in range(_WROWS)]
        for cp in cps:
            cp.wait()
        plsc.subcore_barrier()
        cps = [pltpu.async_copy(aux_hbm.at[idx_v.at[r]], a_v.at[r], sem)
               for r in range(_WROWS)]
        for cp in cps:
            cp.wait()
        plsc.subcore_barrier()

    pltpu.sync_copy(a_v, w_hbm.at[pl.ds(sid * _WROWS, _WROWS)])


_winner = pl.kernel(
    _w_body,
    out_type=[
        jax.ShapeDtypeStruct((B // 128, 128), jnp.int32),
        jax.ShapeDtypeStruct((_AUXN,), jnp.int32),
    ],
    mesh=plsc.VectorSubcoreMesh(
        core_axis_name="c", subcore_axis_name="s", num_cores=1),
    scratch_types=[
        pltpu.VMEM((_WROWS, 128), jnp.int32),
        pltpu.VMEM((_WROWS, 128), jnp.int32),
        pltpu.VMEM((_WROWS, 128), jnp.int32),
        pltpu.VMEM((_WROWS, 128), jnp.int32),
        pltpu.SemaphoreType.DMA,
    ],
)

_SROWS = B // 128 // (NC * NS)


def _s_body(y_hbm, y100_hbm, w_hbm, idx_hbm, tgt_ref, ywin_hbm, wv, iv, rows,
            sem, sem2):
    cid = lax.axis_index("c")
    sid = lax.axis_index("s")
    wid = sid * NC + cid
    rbase = wid * _SROWS

    pltpu.sync_copy(w_hbm.at[pl.ds(rbase, _SROWS)], wv)
    pltpu.sync_copy(idx_hbm.at[pl.ds(rbase, _SROWS)], iv)

    cps = [pltpu.async_copy(y_hbm.at[wv.at[k]], rows.at[k], sem)
           for k in range(_SROWS)]
    for cp in cps:
        cp.wait()
    cps = [pltpu.async_copy(rows.at[k],
                            ywin_hbm.at[pl.ds((rbase + k) * 128, 128)], sem)
           for k in range(_SROWS)]
    cps2 = []
    for k in range(_SROWS):
        for c in range(8):
            ivec = iv[k, pl.ds(c * 16, 16)]
            wvec = wv[k, pl.ds(c * 16, 16)]
            for l in range(16):
                cps2.append(pltpu.async_copy(
                    y100_hbm.at[pl.ds(wvec[l], 1)],
                    tgt_ref.at[pl.ds(ivec[l], 1)], sem2))
    for cp in cps:
        cp.wait()
    for cp in cps2:
        cp.wait()


_scatter = pl.kernel(
    _s_body,
    out_type=jax.ShapeDtypeStruct((B, 128), jnp.float32),
    mesh=plsc.VectorSubcoreMesh(core_axis_name="c", subcore_axis_name="s"),
    scratch_types=[
        pltpu.VMEM((_SROWS, 128), jnp.int32),
        pltpu.VMEM((_SROWS, 128), jnp.int32),
        pltpu.VMEM((_SROWS, 128, 128), jnp.float32),
        pltpu.SemaphoreType.DMA,
        pltpu.SemaphoreType.DMA,
    ],
)

_BLK_C = 2048


def _c_body(ce_ref, y_ref, g_ref, fin_ref, elr_ref, acc_ref):
    i = pl.program_id(0)
    s = jnp.sum(y_ref[...] * g_ref[...], axis=1)
    part = jnp.sum(jnp.log(1.0 - s))

    @pl.when(i == 0)
    def _():
        acc_ref[0] = 0.0

    acc_ref[0] += part

    @pl.when(i == pl.num_programs(0) - 1)
    def _():
        elr = acc_ref[0] / B * LAM
        elr_ref[0, 0] = elr
        fin_ref[0, 0] = -ce_ref[0, 0] / B + elr


_elr_final = pl.pallas_call(
    _c_body,
    grid=(B // _BLK_C,),
    in_specs=[
        pl.BlockSpec((1, 1), lambda i: (0, 0), memory_space=pltpu.SMEM),
        pl.BlockSpec((_BLK_C, 128), lambda i: (i, 0)),
        pl.BlockSpec((_BLK_C, 128), lambda i: (i, 0)),
    ],
    out_specs=[
        pl.BlockSpec((1, 1), lambda i: (0, 0), memory_space=pltpu.SMEM),
        pl.BlockSpec((1, 1), lambda i: (0, 0), memory_space=pltpu.SMEM),
    ],
    out_shape=[
        jax.ShapeDtypeStruct((1, 1), jnp.float32),
        jax.ShapeDtypeStruct((1, 1), jnp.float32),
    ],
    scratch_shapes=[pltpu.SMEM((1,), jnp.float32)],
)


def kernel(index, output, label, target_train):
    idx2d = index.astype(jnp.int32).reshape(B // 128, 128)
    label_r = label.astype(jnp.int32).reshape(B // _BLK_A, 1, _BLK_A)

    y_pred, y100, ce_sum = _softmax_ce(output, label_r)
    w2d, _ = _winner(idx2d)

    tref = jax.new_ref(target_train)
    y_win = _scatter(y_pred, y100, w2d, idx2d, tref)
    new_target = tref[...]

    fin, elr = _elr_final(ce_sum, y_pred, y_win)
    return (fin[0, 0], elr[0, 0], new_target)


# speedup vs baseline: 11.6461x; 11.6461x over previous
import functools

import jax
import jax.numpy as jnp
from jax import lax
from jax.experimental import pallas as pl
from jax.experimental.pallas import tpu as pltpu
from jax.experimental.pallas import tpu_sc as plsc

B = 16384
C = 100
N = 1000000
LAM = 0.01

NC = 2
NS = 16

_BLK_A = 512


def _a_body(o_ref, lbl_ref, y_ref, y100_ref, ce_ref):
    o = o_ref[...]
    m = jnp.max(o, axis=1, keepdims=True)
    e = jnp.exp(o - m)
    s = jnp.sum(e, axis=1, keepdims=True)
    y = jnp.clip(e / s, 0.0001, 1.0 - 0.0001)
    y100_ref[...] = y
    y_ref[...] = jnp.concatenate(
        [y, jnp.zeros((o.shape[0], 128 - C), jnp.float32)], axis=1)
    lbl = lbl_ref[0, 0, :]
    cols = lax.broadcasted_iota(jnp.int32, o.shape, 1)
    pick = jnp.sum(jnp.where(cols == lbl[:, None], o, 0.0), axis=1)
    lse = m[:, 0] + jnp.log(s[:, 0])

    @pl.when(pl.program_id(0) == 0)
    def _():
        ce_ref[0, 0] = 0.0

    ce_ref[0, 0] += jnp.sum(pick - lse)


_softmax_ce = pl.pallas_call(
    _a_body,
    grid=(B // _BLK_A,),
    in_specs=[
        pl.BlockSpec((_BLK_A, C), lambda i: (i, 0)),
        pl.BlockSpec((1, 1, _BLK_A), lambda i: (i, 0, 0)),
    ],
    out_specs=[
        pl.BlockSpec((_BLK_A, 128), lambda i: (i, 0)),
        pl.BlockSpec((_BLK_A, C), lambda i: (i, 0)),
        pl.BlockSpec((1, 1), lambda i: (0, 0), memory_space=pltpu.SMEM),
    ],
    out_shape=[
        jax.ShapeDtypeStruct((B, 128), jnp.float32),
        jax.ShapeDtypeStruct((B, C), jnp.float32),
        jax.ShapeDtypeStruct((1, 1), jnp.float32),
    ],
)

_WROWS = B // 128 // NS
_NROUNDS = 5
_AUXN = N + 128


def _w_body(idx_hbm, w_hbm, aux_sh, idx_v, jb_v, a_v, ip_v, sem):
    sid = lax.axis_index("s")
    base = sid * (_WROWS * 128)

    pltpu.sync_copy(idx_hbm.at[pl.ds(sid * _WROWS, _WROWS)], idx_v)
    for r in range(_WROWS):
        for c in range(8):
            jb_v[r, pl.ds(c * 16, 16)] = (
                base + r * 128 + c * 16
                + lax.broadcasted_iota(jnp.int32, (16,), 0)
            )

    cps = [pltpu.async_copy(jb_v.at[r], aux_sh.at[idx_v.at[r]], sem)
           for r in range(_WROWS)]
    for cp in cps:
        cp.wait()
    plsc.subcore_barrier()
    cps = [pltpu.async_copy(aux_sh.at[idx_v.at[r]], a_v.at[r], sem)
           for r in range(_WROWS)]
    for cp in cps:
        cp.wait()
    plsc.subcore_barrier()

    for _ in range(_NROUNDS):
        for r in range(_WROWS):
            for c in range(8):
                sl = pl.ds(c * 16, 16)
                ip_v[r, sl] = jnp.where(
                    jb_v[r, sl] > a_v[r, sl], idx_v[r, sl], jnp.int32(N))
        cps = [pltpu.async_copy(jb_v.at[r], aux_sh.at[ip_v.at[r]], sem)
               for r in range(_WROWS)]
        for cp in cps:
            cp.wait()
        plsc.subcore_barrier()
        cps = [pltpu.async_copy(aux_sh.at[idx_v.at[r]], a_v.at[r], sem)
               for r in range(_WROWS)]
        for cp in cps:
            cp.wait()
        plsc.subcore_barrier()

    pltpu.sync_copy(a_v, w_hbm.at[pl.ds(sid * _WROWS, _WROWS)])


_winner = pl.kernel(
    _w_body,
    out_type=jax.ShapeDtypeStruct((B // 128, 128), jnp.int32),
    mesh=plsc.VectorSubcoreMesh(
        core_axis_name="c", subcore_axis_name="s", num_cores=1),
    scratch_types=[
        pltpu.VMEM_SHARED((_AUXN,), jnp.int32),
        pltpu.VMEM((_WROWS, 128), jnp.int32),
        pltpu.VMEM((_WROWS, 128), jnp.int32),
        pltpu.VMEM((_WROWS, 128), jnp.int32),
        pltpu.VMEM((_WROWS, 128), jnp.int32),
        pltpu.SemaphoreType.DMA,
    ],
)

_SROWS = B // 128 // (NC * NS)


def _s_body(y_hbm, y100_hbm, w_hbm, idx_hbm, tgt_ref, ywin_hbm, wv, iv, rows,
            sem, sem2):
    cid = lax.axis_index("c")
    sid = lax.axis_index("s")
    wid = sid * NC + cid
    rbase = wid * _SROWS

    pltpu.sync_copy(w_hbm.at[pl.ds(rbase, _SROWS)], wv)
    pltpu.sync_copy(idx_hbm.at[pl.ds(rbase, _SROWS)], iv)

    cps = [pltpu.async_copy(y_hbm.at[wv.at[k]], rows.at[k], sem)
           for k in range(_SROWS)]
    for cp in cps:
        cp.wait()
    cps = [pltpu.async_copy(rows.at[k],
                            ywin_hbm.at[pl.ds((rbase + k) * 128, 128)], sem)
           for k in range(_SROWS)]
    cps2 = []
    for k in range(_SROWS):
        for c in range(8):
            ivec = iv[k, pl.ds(c * 16, 16)]
            wvec = wv[k, pl.ds(c * 16, 16)]
            for l in range(16):
                cps2.append(pltpu.async_copy(
                    y100_hbm.at[pl.ds(wvec[l], 1)],
                    tgt_ref.at[pl.ds(ivec[l], 1)], sem2))
    for cp in cps:
        cp.wait()
    for cp in cps2:
        cp.wait()


_scatter = pl.kernel(
    _s_body,
    out_type=jax.ShapeDtypeStruct((B, 128), jnp.float32),
    mesh=plsc.VectorSubcoreMesh(core_axis_name="c", subcore_axis_name="s"),
    scratch_types=[
        pltpu.VMEM((_SROWS, 128), jnp.int32),
        pltpu.VMEM((_SROWS, 128), jnp.int32),
        pltpu.VMEM((_SROWS, 128, 128), jnp.float32),
        pltpu.SemaphoreType.DMA,
        pltpu.SemaphoreType.DMA,
    ],
)

_BLK_C = 2048


def _c_body(ce_ref, y_ref, g_ref, fin_ref, elr_ref, acc_ref):
    i = pl.program_id(0)
    s = jnp.sum(y_ref[...] * g_ref[...], axis=1)
    part = jnp.sum(jnp.log(1.0 - s))

    @pl.when(i == 0)
    def _():
        acc_ref[0] = 0.0

    acc_ref[0] += part

    @pl.when(i == pl.num_programs(0) - 1)
    def _():
        elr = acc_ref[0] / B * LAM
        elr_ref[0, 0] = elr
        fin_ref[0, 0] = -ce_ref[0, 0] / B + elr


_elr_final = pl.pallas_call(
    _c_body,
    grid=(B // _BLK_C,),
    in_specs=[
        pl.BlockSpec((1, 1), lambda i: (0, 0), memory_space=pltpu.SMEM),
        pl.BlockSpec((_BLK_C, 128), lambda i: (i, 0)),
        pl.BlockSpec((_BLK_C, 128), lambda i: (i, 0)),
    ],
    out_specs=[
        pl.BlockSpec((1, 1), lambda i: (0, 0), memory_space=pltpu.SMEM),
        pl.BlockSpec((1, 1), lambda i: (0, 0), memory_space=pltpu.SMEM),
    ],
    out_shape=[
        jax.ShapeDtypeStruct((1, 1), jnp.float32),
        jax.ShapeDtypeStruct((1, 1), jnp.float32),
    ],
    scratch_shapes=[pltpu.SMEM((1,), jnp.float32)],
)


def kernel(index, output, label, target_train):
    idx2d = index.astype(jnp.int32).reshape(B // 128, 128)
    label_r = label.astype(jnp.int32).reshape(B // _BLK_A, 1, _BLK_A)

    y_pred, y100, ce_sum = _softmax_ce(output, label_r)
    w2d = _winner(idx2d)

    tref = jax.new_ref(target_train)
    y_win = _scatter(y_pred, y100, w2d, idx2d, tref)
    new_target = tref[...]

    fin, elr = _elr_final(ce_sum, y_pred, y_win)
    return (fin[0, 0], elr[0, 0], new_target)


# speedup vs baseline: 11.6550x; 1.0008x over previous
import functools

import jax
import jax.numpy as jnp
from jax import lax
from jax.experimental import pallas as pl
from jax.experimental.pallas import tpu as pltpu
from jax.experimental.pallas import tpu_sc as plsc

B = 16384
C = 100
N = 1000000
LAM = 0.01

NC = 2
NS = 16

_BLK_A = 512


def _a_body(o_ref, lbl_ref, y_ref, y100_ref, ce_ref):
    o = o_ref[...]
    m = jnp.max(o, axis=1, keepdims=True)
    e = jnp.exp(o - m)
    s = jnp.sum(e, axis=1, keepdims=True)
    y = jnp.clip(e / s, 0.0001, 1.0 - 0.0001)
    y100_ref[...] = y
    y_ref[...] = jnp.concatenate(
        [y, jnp.zeros((o.shape[0], 128 - C), jnp.float32)], axis=1)
    lbl = lbl_ref[0, 0, :]
    cols = lax.broadcasted_iota(jnp.int32, o.shape, 1)
    pick = jnp.sum(jnp.where(cols == lbl[:, None], o, 0.0), axis=1)
    lse = m[:, 0] + jnp.log(s[:, 0])

    @pl.when(pl.program_id(0) == 0)
    def _():
        ce_ref[0, 0] = 0.0

    ce_ref[0, 0] += jnp.sum(pick - lse)


_softmax_ce = pl.pallas_call(
    _a_body,
    grid=(B // _BLK_A,),
    in_specs=[
        pl.BlockSpec((_BLK_A, C), lambda i: (i, 0)),
        pl.BlockSpec((1, 1, _BLK_A), lambda i: (i, 0, 0)),
    ],
    out_specs=[
        pl.BlockSpec((_BLK_A, 128), lambda i: (i, 0)),
        pl.BlockSpec((_BLK_A, C), lambda i: (i, 0)),
        pl.BlockSpec((1, 1), lambda i: (0, 0), memory_space=pltpu.SMEM),
    ],
    out_shape=[
        jax.ShapeDtypeStruct((B, 128), jnp.float32),
        jax.ShapeDtypeStruct((B, C), jnp.float32),
        jax.ShapeDtypeStruct((1, 1), jnp.float32),
    ],
)

_WROWS = B // 128 // NS
_NROUNDS = 5
_AUXN = N + 128


def _w_body(idx_hbm, w_hbm, aux_sh, idx_v, jb_v, a_v, ip_v, sem):
    sid = lax.axis_index("s")
    base = sid * (_WROWS * 128)

    pltpu.sync_copy(idx_hbm.at[pl.ds(sid * _WROWS, _WROWS)], idx_v)
    for r in range(_WROWS):
        for c in range(8):
            jb_v[r, pl.ds(c * 16, 16)] = (
                base + r * 128 + c * 16
                + lax.broadcasted_iota(jnp.int32, (16,), 0)
            )

    cps = [pltpu.async_copy(jb_v.at[r], aux_sh.at[idx_v.at[r]], sem)
           for r in range(_WROWS)]
    for cp in cps:
        cp.wait()
    plsc.subcore_barrier()
    cps = [pltpu.async_copy(aux_sh.at[idx_v.at[r]], a_v.at[r], sem)
           for r in range(_WROWS)]
    for cp in cps:
        cp.wait()
    plsc.subcore_barrier()

    for _ in range(_NROUNDS):
        for r in range(_WROWS):
            for c in range(8):
                sl = pl.ds(c * 16, 16)
                ip_v[r, sl] = jnp.where(
                    jb_v[r, sl] > a_v[r, sl], idx_v[r, sl], jnp.int32(N))
        cps = [pltpu.async_copy(jb_v.at[r], aux_sh.at[ip_v.at[r]], sem)
               for r in range(_WROWS)]
        for cp in cps:
            cp.wait()
        plsc.subcore_barrier()
        cps = [pltpu.async_copy(aux_sh.at[idx_v.at[r]], a_v.at[r], sem)
               for r in range(_WROWS)]
        for cp in cps:
            cp.wait()
        plsc.subcore_barrier()

    pltpu.sync_copy(a_v, w_hbm.at[pl.ds(sid * _WROWS, _WROWS)])


_winner = pl.kernel(
    _w_body,
    out_type=jax.ShapeDtypeStruct((B // 128, 128), jnp.int32),
    mesh=plsc.VectorSubcoreMesh(
        core_axis_name="c", subcore_axis_name="s", num_cores=1),
    scratch_types=[
        pltpu.VMEM_SHARED((_AUXN,), jnp.int32),
        pltpu.VMEM((_WROWS, 128), jnp.int32),
        pltpu.VMEM((_WROWS, 128), jnp.int32),
        pltpu.VMEM((_WROWS, 128), jnp.int32),
        pltpu.VMEM((_WROWS, 128), jnp.int32),
        pltpu.SemaphoreType.DMA,
    ],
)

_SROWS = B // 128 // (NC * NS)


def _s_body(y_hbm, y100_hbm, w_hbm, idx_hbm, tgt_ref, ywin_hbm, wv, iv, rows,
            sem, sem2):
    cid = lax.axis_index("c")
    sid = lax.axis_index("s")
    wid = sid * NC + cid
    rbase = wid * _SROWS

    pltpu.sync_copy(w_hbm.at[pl.ds(rbase, _SROWS)], wv)
    pltpu.sync_copy(idx_hbm.at[pl.ds(rbase, _SROWS)], iv)

    cps = [pltpu.async_copy(y_hbm.at[wv.at[k]], rows.at[k], sem)
           for k in range(_SROWS)]
    for cp in cps:
        cp.wait()
    cps = [pltpu.async_copy(rows.at[k],
                            ywin_hbm.at[pl.ds((rbase + k) * 128, 128)], sem)
           for k in range(_SROWS)]
    cps2 = []
    for k in range(_SROWS):
        for c in range(8):
            ivec = iv[k, pl.ds(c * 16, 16)]
            wvec = wv[k, pl.ds(c * 16, 16)]
            for l in range(16):
                cps2.append(pltpu.async_copy(
                    y100_hbm.at[pl.ds(wvec[l], 1)],
                    tgt_ref.at[pl.ds(ivec[l], 1)], sem2))
    for cp in cps:
        cp.wait()
    for cp in cps2:
        cp.wait()


_scatter = pl.kernel(
    _s_body,
    out_type=jax.ShapeDtypeStruct((B, 128), jnp.float32),
    mesh=plsc.VectorSubcoreMesh(core_axis_name="c", subcore_axis_name="s"),
    scratch_types=[
        pltpu.VMEM((_SROWS, 128), jnp.int32),
        pltpu.VMEM((_SROWS, 128), jnp.int32),
        pltpu.VMEM((_SROWS, 128, 128), jnp.float32),
        pltpu.SemaphoreType.DMA,
        pltpu.SemaphoreType.DMA,
    ],
)

_BLK_C = 2048


def _c_body(ce_ref, y_ref, g_ref, fin_ref, elr_ref, acc_ref):
    i = pl.program_id(0)
    s = jnp.sum(y_ref[...] * g_ref[...], axis=1)
    part = jnp.sum(jnp.log(1.0 - s))

    @pl.when(i == 0)
    def _():
        acc_ref[0] = 0.0

    acc_ref[0] += part

    @pl.when(i == pl.num_programs(0) - 1)
    def _():
        elr = acc_ref[0] / B * LAM
        elr_ref[0, 0] = elr
        fin_ref[0, 0] = -ce_ref[0, 0] / B + elr


_elr_final = pl.pallas_call(
    _c_body,
    grid=(B // _BLK_C,),
    in_specs=[
        pl.BlockSpec((1, 1), lambda i: (0, 0), memory_space=pltpu.SMEM),
        pl.BlockSpec((_BLK_C, 128), lambda i: (i, 0)),
        pl.BlockSpec((_BLK_C, 128), lambda i: (i, 0)),
    ],
    out_specs=[
        pl.BlockSpec((1, 1), lambda i: (0, 0), memory_space=pltpu.SMEM),
        pl.BlockSpec((1, 1), lambda i: (0, 0), memory_space=pltpu.SMEM),
    ],
    out_shape=[
        jax.ShapeDtypeStruct((1, 1), jnp.float32),
        jax.ShapeDtypeStruct((1, 1), jnp.float32),
    ],
    scratch_shapes=[pltpu.SMEM((1,), jnp.float32)],
)


def kernel(index, output, label, target_train):
    idx2d = index.astype(jnp.int32).reshape(B // 128, 128)
    label_r = label.astype(jnp.int32).reshape(B // _BLK_A, 1, _BLK_A)

    y_pred, y100, ce_sum = _softmax_ce(output, label_r)
    w2d = _winner(idx2d)

    tref = jax.new_ref(target_train)
    y_win = _scatter(y_pred, y100, w2d, idx2d, tref)
    new_target = jax.freeze(tref)

    fin, elr = _elr_final(ce_sum, y_pred, y_win)
    return (fin[0, 0], elr[0, 0], new_target)
